# baseline (speedup 1.0000x reference)
import functools

import jax
import jax.numpy as jnp
from jax import lax
from jax.experimental import pallas as pl
from jax.experimental.pallas import tpu as pltpu
from jax.experimental.pallas import tpu_sc as plsc

HIDDEN_DIM = 32
MAX_NORM = 1.0
VOCAB = 1000000

B_ROWS = 16384 * 26
NC, NS, LANES = 2, 16, 16
NW = NC * NS

_mesh = plsc.VectorSubcoreMesh(core_axis_name="c", subcore_axis_name="s")

TCOLS = 512
OROWS = TCOLS // 4
FULL_BLOCKS = 1952
BLK_PER_W = FULL_BLOCKS // NW


@functools.partial(
    pl.kernel,
    out_type=jax.ShapeDtypeStruct((VOCAB // 4, 128), jnp.float32),
    mesh=_mesh,
    compiler_params=pltpu.CompilerParams(needs_layout_passes=False,
                                         use_tc_tiling_on_sc=True),
    scratch_types=[
        pltpu.VMEM((HIDDEN_DIM, TCOLS + 1), jnp.float32),
        pltpu.VMEM((HIDDEN_DIM, TCOLS + 1), jnp.float32),
        pltpu.VMEM((OROWS, 128), jnp.float32),
        pltpu.VMEM((OROWS, 128), jnp.float32),
        pltpu.SemaphoreType.DMA,
        pltpu.SemaphoreType.DMA,
        pltpu.SemaphoreType.DMA,
        pltpu.SemaphoreType.DMA,
    ],
)
def _transpose_sc(tt_hbm, tail4_hbm, t4_hbm, ibuf_a, ibuf_b, obuf_a, obuf_b,
                  sem_la, sem_lb, sem_sa, sem_sb):
    wid = lax.axis_index("s") * NC + lax.axis_index("c")

    def col0(b):
        return pl.multiple_of((wid * BLK_PER_W + b) * TCOLS, TCOLS)

    def load_desc(b, ibuf, sem, ncols=TCOLS):
        return pltpu.make_async_copy(
            tt_hbm.at[pl.ds(0, HIDDEN_DIM), pl.ds(col0(b), ncols)],
            ibuf.at[pl.ds(0, HIDDEN_DIM), pl.ds(0, ncols)], sem)

    def store_desc(b, obuf, sem, nrows=OROWS):
        return pltpu.make_async_copy(
            obuf.at[pl.ds(0, nrows)],
            t4_hbm.at[pl.ds(pl.multiple_of(col0(b) // 4, OROWS), nrows)],
            sem)

    dvec0 = lax.iota(jnp.int32, LANES)
    dvec1 = dvec0 + LANES

    def do_block(ibuf, obuf, nrows=OROWS):
        @pl.loop(0, nrows, unroll=8)
        def _row(i):
            cb = i * 4
            for k in range(8):
                dvec = dvec1 if (k % 2) else dvec0
                col = jnp.full((LANES,), 0, jnp.int32) + (cb + k // 2)
                obuf[i, pl.ds(k * LANES, LANES)] = plsc.load_gather(
                    ibuf, [dvec, col])

    bufs = ((ibuf_a, obuf_a, sem_la, sem_sa),
            (ibuf_b, obuf_b, sem_lb, sem_sb))

    load_desc(0, ibuf_a, sem_la).start()
    load_desc(1, ibuf_b, sem_lb).start()

    @pl.loop(0, BLK_PER_W // 2)
    def _super(s):
        for p, (ibuf, obuf, sem_l, sem_s) in enumerate(bufs):
            b = s * 2 + p

            @pl.when(s >= 1)
            def _():
                store_desc(b - 2, obuf, sem_s).wait()

            load_desc(b, ibuf, sem_l).wait()
            do_block(ibuf, obuf)
            store_desc(b, obuf, sem_s).start()

            @pl.when(b + 2 < BLK_PER_W)
            def _():
                load_desc(b + 2, ibuf, sem_l).start()

    b_last = BLK_PER_W - 1
    load_desc(b_last, ibuf_a, sem_la).wait()
    store_desc(b_last - 2, obuf_a, sem_sa).wait()
    do_block(ibuf_a, obuf_a)
    store_desc(b_last, obuf_a, sem_sa).start()
    store_desc(b_last - 1, obuf_b, sem_sb).wait()
    store_desc(b_last, obuf_a, sem_sa).wait()

    @pl.when(wid == 0)
    def _():
        pltpu.make_async_copy(
            tt_hbm.at[pl.ds(0, HIDDEN_DIM), pl.ds(FULL_BLOCKS * TCOLS,
                                                  TCOLS)],
            ibuf_b.at[pl.ds(0, HIDDEN_DIM), pl.ds(0, TCOLS)],
            sem_lb).start()
        pltpu.make_async_copy(
            tt_hbm.at[pl.ds(0, HIDDEN_DIM), pl.ds(FULL_BLOCKS * TCOLS,
                                                  TCOLS)],
            ibuf_b.at[pl.ds(0, HIDDEN_DIM), pl.ds(0, TCOLS)],
            sem_lb).wait()
        do_block(ibuf_b, obuf_b)
        pltpu.make_async_copy(
            obuf_b, t4_hbm.at[pl.ds(FULL_BLOCKS * TCOLS // 4, OROWS)],
            sem_sb).start()
        pltpu.make_async_copy(
            obuf_b, t4_hbm.at[pl.ds(FULL_BLOCKS * TCOLS // 4, OROWS)],
            sem_sb).wait()

    @pl.when(wid == NW - 1)
    def _():
        pltpu.sync_copy(tail4_hbm, obuf_b.at[pl.ds(0, 16)])
        pltpu.sync_copy(obuf_b.at[pl.ds(0, 16)],
                        t4_hbm.at[pl.ds((VOCAB - 64) // 4, 16)])


ROWS_PER_W = B_ROWS // NW
CHUNK = 512
GROUPS = CHUNK // 128
NCHUNKS = ROWS_PER_W // CHUNK
IDX_ROWS = ROWS_PER_W // 128


@functools.partial(
    pl.kernel,
    out_type=jax.ShapeDtypeStruct((B_ROWS, HIDDEN_DIM), jnp.float32),
    mesh=_mesh,
    compiler_params=pltpu.CompilerParams(needs_layout_passes=False,
                                         use_tc_tiling_on_sc=False),
    scratch_types=[
        pltpu.VMEM((IDX_ROWS, 128), jnp.int32),
        pltpu.VMEM((CHUNK, HIDDEN_DIM), jnp.float32),
        pltpu.VMEM((CHUNK, HIDDEN_DIM), jnp.float32),
        pltpu.VMEM((CHUNK, HIDDEN_DIM), jnp.float32),
        pltpu.VMEM((CHUNK, HIDDEN_DIM), jnp.float32),
        pltpu.SemaphoreType.DMA,
        pltpu.SemaphoreType.DMA,
        pltpu.SemaphoreType.DMA,
        pltpu.SemaphoreType.DMA,
    ],
)
def _hubs_sc(x_hbm, tbl_hbm, out_hbm, idx_all,
             rows_a, rows_b, outv_a, outv_b,
             sem_ga, sem_gb, sem_oa, sem_ob):
    wid = lax.axis_index("s") * NC + lax.axis_index("c")
    base = wid * ROWS_PER_W

    pltpu.sync_copy(x_hbm.at[pl.ds(pl.multiple_of(wid * IDX_ROWS, 8),
                                   IDX_ROWS)], idx_all)

    def fire_gathers(c, rows, sem):
        for j in range(GROUPS):
            pltpu.async_copy(tbl_hbm.at[idx_all.at[c * GROUPS + j]],
                             rows.at[pl.ds(j * 128, 128)], sem)

    def wait_gathers(c, rows, sem):
        for j in range(GROUPS):
            pltpu.make_async_copy(tbl_hbm.at[idx_all.at[c * GROUPS + j]],
                                  rows.at[pl.ds(j * 128, 128)], sem).wait()

    def out_slice(c):
        return out_hbm.at[pl.ds(base + c * CHUNK, CHUNK)]

    bufs = ((rows_a, outv_a, sem_ga, sem_oa),
            (rows_b, outv_b, sem_gb, sem_ob))

    fire_gathers(0, rows_a, sem_ga)
    fire_gathers(1, rows_b, sem_gb)

    @pl.loop(0, NCHUNKS // 2)
    def _super(s):
        for p, (rows, outv, sem_g, sem_o) in enumerate(bufs):
            c = s * 2 + p

            @pl.when(s >= 1)
            def _():
                pltpu.make_async_copy(outv, out_slice(c - 2), sem_o).wait()

            wait_gathers(c, rows, sem_g)

            @pl.loop(0, CHUNK // LANES, unroll=2)
            def _blk(i):
                lane = lax.iota(jnp.int32, LANES)
                row_idx = i * LANES + lane
                acc = [jnp.zeros((LANES,), jnp.float32) for _ in range(4)]
                vals = []
                for d in range(HIDDEN_DIM):
                    colv = (lane + d) & (HIDDEN_DIM - 1)
                    vd = plsc.load_gather(rows, [row_idx, colv])
                    vals.append(vd)
                    acc[d % 4] = acc[d % 4] + vd * vd
                ss = (acc[0] + acc[1]) + (acc[2] + acc[3])
                bi = plsc.bitcast(ss, jnp.int32)
                y = plsc.bitcast(jnp.int32(0x5F3759DF) - (bi >> 1),
                                 jnp.float32)
                y = y * (1.5 - 0.5 * ss * y * y)
                y = y * (1.5 - 0.5 * ss * y * y)
                scale = jnp.where(ss > MAX_NORM * MAX_NORM, y,
                                  jnp.float32(1.0))
                for d in range(HIDDEN_DIM):
                    colv = (lane + d) & (HIDDEN_DIM - 1)
                    plsc.store_scatter(outv, [row_idx, colv],
                                       vals[d] * scale)

            pltpu.async_copy(outv, out_slice(c), sem_o)

            @pl.when(s < NCHUNKS // 2 - 1)
            def _():
                fire_gathers(c + 2, rows, sem_g)

    pltpu.make_async_copy(outv_a, out_slice(NCHUNKS - 2), sem_oa).wait()
    pltpu.make_async_copy(outv_b, out_slice(NCHUNKS - 1), sem_ob).wait()


def kernel(x, table):
    x2d = x.reshape(-1).astype(jnp.int32).reshape(B_ROWS // 128, 128)
    out = _hubs_sc(x2d, table)
    return out.reshape(x.shape[0], x.shape[1], HIDDEN_DIM)


# speedup vs baseline: 1.0369x; 1.0369x over previous
import functools

import jax
import jax.numpy as jnp
from jax import lax
from jax.experimental import pallas as pl
from jax.experimental.pallas import tpu as pltpu
from jax.experimental.pallas import tpu_sc as plsc

HIDDEN_DIM = 32
MAX_NORM = 1.0
VOCAB = 1000000

B_ROWS = 16384 * 26
NC, NS, LANES = 2, 16, 16
NW = NC * NS

_mesh = plsc.VectorSubcoreMesh(core_axis_name="c", subcore_axis_name="s")

TCOLS = 512
OROWS = TCOLS // 4
FULL_BLOCKS = 1952
BLK_PER_W = FULL_BLOCKS // NW


@functools.partial(
    pl.kernel,
    out_type=jax.ShapeDtypeStruct((VOCAB // 4, 128), jnp.float32),
    mesh=_mesh,
    compiler_params=pltpu.CompilerParams(needs_layout_passes=False,
                                         use_tc_tiling_on_sc=True),
    scratch_types=[
        pltpu.VMEM((HIDDEN_DIM, TCOLS + 1), jnp.float32),
        pltpu.VMEM((HIDDEN_DIM, TCOLS + 1), jnp.float32),
        pltpu.VMEM((OROWS, 128), jnp.float32),
        pltpu.VMEM((OROWS, 128), jnp.float32),
        pltpu.SemaphoreType.DMA,
        pltpu.SemaphoreType.DMA,
        pltpu.SemaphoreType.DMA,
        pltpu.SemaphoreType.DMA,
    ],
)
def _transpose_sc(tt_hbm, tail4_hbm, t4_hbm, ibuf_a, ibuf_b, obuf_a, obuf_b,
                  sem_la, sem_lb, sem_sa, sem_sb):
    wid = lax.axis_index("s") * NC + lax.axis_index("c")

    def col0(b):
        return pl.multiple_of((wid * BLK_PER_W + b) * TCOLS, TCOLS)

    def load_desc(b, ibuf, sem, ncols=TCOLS):
        return pltpu.make_async_copy(
            tt_hbm.at[pl.ds(0, HIDDEN_DIM), pl.ds(col0(b), ncols)],
            ibuf.at[pl.ds(0, HIDDEN_DIM), pl.ds(0, ncols)], sem)

    def store_desc(b, obuf, sem, nrows=OROWS):
        return pltpu.make_async_copy(
            obuf.at[pl.ds(0, nrows)],
            t4_hbm.at[pl.ds(pl.multiple_of(col0(b) // 4, OROWS), nrows)],
            sem)

    dvec0 = lax.iota(jnp.int32, LANES)
    dvec1 = dvec0 + LANES

    def do_block(ibuf, obuf, nrows=OROWS):
        @pl.loop(0, nrows, unroll=8)
        def _row(i):
            cb = i * 4
            for k in range(8):
                dvec = dvec1 if (k % 2) else dvec0
                col = jnp.full((LANES,), 0, jnp.int32) + (cb + k // 2)
                obuf[i, pl.ds(k * LANES, LANES)] = plsc.load_gather(
                    ibuf, [dvec, col])

    bufs = ((ibuf_a, obuf_a, sem_la, sem_sa),
            (ibuf_b, obuf_b, sem_lb, sem_sb))

    load_desc(0, ibuf_a, sem_la).start()
    load_desc(1, ibuf_b, sem_lb).start()

    @pl.loop(0, BLK_PER_W // 2)
    def _super(s):
        for p, (ibuf, obuf, sem_l, sem_s) in enumerate(bufs):
            b = s * 2 + p

            @pl.when(s >= 1)
            def _():
                store_desc(b - 2, obuf, sem_s).wait()

            load_desc(b, ibuf, sem_l).wait()
            do_block(ibuf, obuf)
            store_desc(b, obuf, sem_s).start()

            @pl.when(b + 2 < BLK_PER_W)
            def _():
                load_desc(b + 2, ibuf, sem_l).start()

    b_last = BLK_PER_W - 1
    load_desc(b_last, ibuf_a, sem_la).wait()
    store_desc(b_last - 2, obuf_a, sem_sa).wait()
    do_block(ibuf_a, obuf_a)
    store_desc(b_last, obuf_a, sem_sa).start()
    store_desc(b_last - 1, obuf_b, sem_sb).wait()
    store_desc(b_last, obuf_a, sem_sa).wait()

    @pl.when(wid == 0)
    def _():
        pltpu.make_async_copy(
            tt_hbm.at[pl.ds(0, HIDDEN_DIM), pl.ds(FULL_BLOCKS * TCOLS,
                                                  TCOLS)],
            ibuf_b.at[pl.ds(0, HIDDEN_DIM), pl.ds(0, TCOLS)],
            sem_lb).start()
        pltpu.make_async_copy(
            tt_hbm.at[pl.ds(0, HIDDEN_DIM), pl.ds(FULL_BLOCKS * TCOLS,
                                                  TCOLS)],
            ibuf_b.at[pl.ds(0, HIDDEN_DIM), pl.ds(0, TCOLS)],
            sem_lb).wait()
        do_block(ibuf_b, obuf_b)
        pltpu.make_async_copy(
            obuf_b, t4_hbm.at[pl.ds(FULL_BLOCKS * TCOLS // 4, OROWS)],
            sem_sb).start()
        pltpu.make_async_copy(
            obuf_b, t4_hbm.at[pl.ds(FULL_BLOCKS * TCOLS // 4, OROWS)],
            sem_sb).wait()

    @pl.when(wid == NW - 1)
    def _():
        pltpu.sync_copy(tail4_hbm, obuf_b.at[pl.ds(0, 16)])
        pltpu.sync_copy(obuf_b.at[pl.ds(0, 16)],
                        t4_hbm.at[pl.ds((VOCAB - 64) // 4, 16)])


ROWS_PER_W = B_ROWS // NW
CHUNK = 512
GROUPS = CHUNK // 128
NCHUNKS = ROWS_PER_W // CHUNK
IDX_ROWS = ROWS_PER_W // 128


@functools.partial(
    pl.kernel,
    out_type=jax.ShapeDtypeStruct((B_ROWS, HIDDEN_DIM), jnp.float32),
    mesh=_mesh,
    compiler_params=pltpu.CompilerParams(needs_layout_passes=False,
                                         use_tc_tiling_on_sc=False),
    scratch_types=[
        pltpu.VMEM((IDX_ROWS, 128), jnp.int32),
        pltpu.VMEM((CHUNK, HIDDEN_DIM), jnp.float32),
        pltpu.VMEM((CHUNK, HIDDEN_DIM), jnp.float32),
        pltpu.VMEM((CHUNK, HIDDEN_DIM), jnp.float32),
        pltpu.VMEM((CHUNK, HIDDEN_DIM), jnp.float32),
        pltpu.SemaphoreType.DMA,
        pltpu.SemaphoreType.DMA,
        pltpu.SemaphoreType.DMA,
        pltpu.SemaphoreType.DMA,
    ],
)
def _hubs_sc(x_hbm, tbl_hbm, out_hbm, idx_all,
             rows_a, rows_b, outv_a, outv_b,
             sem_ga, sem_gb, sem_oa, sem_ob):
    wid = lax.axis_index("s") * NC + lax.axis_index("c")
    base = wid * ROWS_PER_W

    pltpu.sync_copy(x_hbm.at[pl.ds(pl.multiple_of(wid * IDX_ROWS, 8),
                                   IDX_ROWS)], idx_all)

    def fire_gathers(c, rows, sem):
        for j in range(GROUPS):
            pltpu.async_copy(tbl_hbm.at[idx_all.at[c * GROUPS + j]],
                             rows.at[pl.ds(j * 128, 128)], sem)

    def wait_gathers(c, rows, sem):
        for j in range(GROUPS):
            pltpu.make_async_copy(tbl_hbm.at[idx_all.at[c * GROUPS + j]],
                                  rows.at[pl.ds(j * 128, 128)], sem).wait()

    def out_slice(c):
        return out_hbm.at[pl.ds(base + c * CHUNK, CHUNK)]

    bufs = ((rows_a, outv_a, sem_ga, sem_oa),
            (rows_b, outv_b, sem_gb, sem_ob))

    fire_gathers(0, rows_a, sem_ga)
    fire_gathers(1, rows_b, sem_gb)

    @pl.loop(0, NCHUNKS // 2)
    def _super(s):
        for p, (rows, outv, sem_g, sem_o) in enumerate(bufs):
            c = s * 2 + p

            @pl.when(s >= 1)
            def _():
                pltpu.make_async_copy(outv, out_slice(c - 2), sem_o).wait()

            wait_gathers(c, rows, sem_g)

            @pl.loop(0, CHUNK // LANES, unroll=1)
            def _blk(i):
                lane = lax.iota(jnp.int32, LANES)
                row_idx = i * LANES + lane
                acc = [jnp.zeros((LANES,), jnp.float32) for _ in range(4)]
                vals = []
                for d in range(HIDDEN_DIM):
                    colv = (lane + d) & (HIDDEN_DIM - 1)
                    vd = plsc.load_gather(rows, [row_idx, colv])
                    vals.append(vd)
                    acc[d % 4] = acc[d % 4] + vd * vd
                ss = (acc[0] + acc[1]) + (acc[2] + acc[3])
                bi = plsc.bitcast(ss, jnp.int32)
                y = plsc.bitcast(jnp.int32(0x5F3759DF) - (bi >> 1),
                                 jnp.float32)
                y = y * (1.5 - 0.5 * ss * y * y)
                y = y * (1.5 - 0.5 * ss * y * y)
                scale = jnp.where(ss > MAX_NORM * MAX_NORM, y,
                                  jnp.float32(1.0))
                for d in range(HIDDEN_DIM):
                    colv = (lane + d) & (HIDDEN_DIM - 1)
                    plsc.store_scatter(outv, [row_idx, colv],
                                       vals[d] * scale)

            pltpu.async_copy(outv, out_slice(c), sem_o)

            @pl.when(s < NCHUNKS // 2 - 1)
            def _():
                fire_gathers(c + 2, rows, sem_g)

    pltpu.make_async_copy(outv_a, out_slice(NCHUNKS - 2), sem_oa).wait()
    pltpu.make_async_copy(outv_b, out_slice(NCHUNKS - 1), sem_ob).wait()


def kernel(x, table):
    x2d = x.reshape(-1).astype(jnp.int32).reshape(B_ROWS // 128, 128)
    out = _hubs_sc(x2d, table)
    return out.reshape(x.shape[0], x.shape[1], HIDDEN_DIM)
